# Initial kernel scaffold; baseline (speedup 1.0000x reference)
import jax
import jax.numpy as jnp
from jax.experimental import pallas as pl


def kernel(x, batch, conv1_w, conv1_b, convs_w, convs_b, bn_gamma, bn_beta, lin_w, lin_b, out_w, out_b):
    raise NotImplementedError("write your pallas kernel here")


# speedup vs baseline: 5.0718x; 5.0718x over previous
import functools

import jax
import jax.numpy as jnp
from jax.experimental import pallas as pl

N = 4096
K_NN = 100
B = 8
RB = 256
PB = 512

_HIGH = jax.lax.Precision.HIGHEST


def _lrelu(v):
    return jnp.where(v >= 0, v, 0.01 * v)


def _graph_kernel(xr_ref, xc_ref, br_ref, bc_ref, coef_ref, dr_ref, dc_ref, m_ref):
    i = pl.program_id(0)
    xr = xr_ref[...]
    xc = xc_ref[...]
    d2r = jnp.sum(xr * xr, axis=1, keepdims=True)
    d2c = jnp.sum(xc * xc, axis=1, keepdims=True).T
    cross = jax.lax.dot_general(xr, xc, (((1,), (1,)), ((), ())),
                                preferred_element_type=jnp.float32)
    dist = d2r + d2c - 2.0 * cross

    iota_c = jax.lax.broadcasted_iota(jnp.int32, (RB, N), 1)
    rows = i * RB + jax.lax.broadcasted_iota(jnp.int32, (RB, 1), 0)
    same = br_ref[...] == bc_ref[...]
    valid = same & (iota_c != rows)
    inf = jnp.float32(jnp.inf)
    dist = jnp.where(valid, dist, inf)

    coef = coef_ref[...]
    iota_k = jax.lax.broadcasted_iota(jnp.int32, (RB, K_NN), 1)

    def body(j, carry):
        d, macc = carry
        m = jnp.min(d, axis=1, keepdims=True)
        cand = jnp.where(d == m, iota_c, N)
        amin = jnp.min(cand, axis=1, keepdims=True)
        onehot = iota_c == amin
        cj = jnp.sum(jnp.where(iota_k == j, coef, 0.0), axis=1, keepdims=True)
        macc = macc + jnp.where(onehot, cj, 0.0)
        d = jnp.where(onehot, inf, d)
        return d, macc

    macc = jnp.zeros((RB, N), jnp.float32)
    _, macc = jax.lax.fori_loop(0, K_NN, body, (dist, macc))

    self_onehot = iota_c == rows
    macc = macc + jnp.where(self_onehot, dr_ref[...], 0.0)
    m_ref[...] = macc * dc_ref[...]


def _build_m(x, batch_r, batch_c, coef, dinv_r, dinv_c):
    grid = (N // RB,)
    return pl.pallas_call(
        _graph_kernel,
        grid=grid,
        in_specs=[
            pl.BlockSpec((RB, 32), lambda i: (i, 0)),
            pl.BlockSpec((N, 32), lambda i: (0, 0)),
            pl.BlockSpec((RB, 1), lambda i: (i, 0)),
            pl.BlockSpec((1, N), lambda i: (0, 0)),
            pl.BlockSpec((RB, K_NN), lambda i: (i, 0)),
            pl.BlockSpec((RB, 1), lambda i: (i, 0)),
            pl.BlockSpec((1, N), lambda i: (0, 0)),
        ],
        out_specs=pl.BlockSpec((RB, N), lambda i: (i, 0)),
        out_shape=jax.ShapeDtypeStruct((N, N), jnp.float32),
    )(x, x, batch_r, batch_c, coef, dinv_r, dinv_c)


def _prop_kernel(m_ref, h_ref, o_ref):
    o_ref[...] = jax.lax.dot_general(
        m_ref[...], h_ref[...], (((1,), (0,)), ((), ())),
        precision=_HIGH, preferred_element_type=jnp.float32)


def _prop(m, h):
    f = h.shape[1]
    return pl.pallas_call(
        _prop_kernel,
        grid=(N // PB,),
        in_specs=[
            pl.BlockSpec((PB, N), lambda i: (i, 0)),
            pl.BlockSpec((N, f), lambda i: (0, 0)),
        ],
        out_specs=pl.BlockSpec((PB, f), lambda i: (i, 0)),
        out_shape=jax.ShapeDtypeStruct((N, f), jnp.float32),
    )(m, h)


def _prop_dense_kernel(m_ref, h_ref, w_ref, b_ref, o_ref):
    t = jax.lax.dot_general(m_ref[...], h_ref[...], (((1,), (0,)), ((), ())),
                            precision=_HIGH, preferred_element_type=jnp.float32)
    z = jax.lax.dot_general(t, w_ref[...], (((1,), (0,)), ((), ())),
                            precision=_HIGH, preferred_element_type=jnp.float32)
    o_ref[...] = _lrelu(z + b_ref[...])


def _prop_dense(m, h, w, b):
    f_in = h.shape[1]
    f_out = w.shape[1]
    return pl.pallas_call(
        _prop_dense_kernel,
        grid=(N // PB,),
        in_specs=[
            pl.BlockSpec((PB, N), lambda i: (i, 0)),
            pl.BlockSpec((N, f_in), lambda i: (0, 0)),
            pl.BlockSpec((f_in, f_out), lambda i: (0, 0)),
            pl.BlockSpec((1, f_out), lambda i: (0, 0)),
        ],
        out_specs=pl.BlockSpec((PB, f_out), lambda i: (i, 0)),
        out_shape=jax.ShapeDtypeStruct((N, f_out), jnp.float32),
    )(m, h, w, b)


def _pool_kernel(h_ref, b_ref, o_ref):
    h = h_ref[...]
    bt = b_ref[...]
    ninf = jnp.float32(-jnp.inf)
    rows = []
    for g in range(B):
        mask = bt == g
        s = jnp.sum(jnp.where(mask, h, 0.0), axis=0)
        cnt = jnp.sum(jnp.where(mask, 1.0, 0.0))
        mean = s / jnp.maximum(cnt, 1.0)
        mx = jnp.max(jnp.where(mask, h, ninf), axis=0)
        mx = jnp.where(jnp.isfinite(mx), mx, 0.0)
        rows.append(jnp.concatenate([mean, mx]))
    o_ref[...] = jnp.stack(rows)


def _pool(h, batch_r):
    f = h.shape[1]
    return pl.pallas_call(
        _pool_kernel,
        in_specs=[
            pl.BlockSpec((N, f), lambda: (0, 0)),
            pl.BlockSpec((N, 1), lambda: (0, 0)),
        ],
        out_specs=pl.BlockSpec((B, 2 * f), lambda: (0, 0)),
        out_shape=jax.ShapeDtypeStruct((B, 2 * f), jnp.float32),
    )(h, batch_r)


def _mlp_kernel(z_ref, g_ref, be_ref, lw_ref, lb_ref, ow_ref, ob_ref, o_ref):
    z = z_ref[...]
    mu = jnp.mean(z, axis=0, keepdims=True)
    var = jnp.mean((z - mu) ** 2, axis=0, keepdims=True)
    z = (z - mu) / jnp.sqrt(var + 1e-5) * g_ref[...] + be_ref[...]
    for g in range(lw_ref.shape[0]):
        z = _lrelu(jax.lax.dot_general(
            z, lw_ref[g], (((1,), (0,)), ((), ())),
            precision=_HIGH, preferred_element_type=jnp.float32) + lb_ref[g][None, :])
    o_ref[...] = jax.lax.dot_general(
        z, ow_ref[...], (((1,), (0,)), ((), ())),
        precision=_HIGH, preferred_element_type=jnp.float32) + ob_ref[...]


def _mlp(z, gamma, beta, lin_w, lin_b, out_w, out_b):
    f = z.shape[1]
    nl = out_w.shape[1]
    return pl.pallas_call(
        _mlp_kernel,
        in_specs=[
            pl.BlockSpec((B, f), lambda: (0, 0)),
            pl.BlockSpec((1, f), lambda: (0, 0)),
            pl.BlockSpec((1, f), lambda: (0, 0)),
            pl.BlockSpec(lin_w.shape, lambda: (0, 0, 0)),
            pl.BlockSpec(lin_b.shape, lambda: (0, 0)),
            pl.BlockSpec(out_w.shape, lambda: (0, 0)),
            pl.BlockSpec((1, nl), lambda: (0, 0)),
        ],
        out_specs=pl.BlockSpec((B, nl), lambda: (0, 0)),
        out_shape=jax.ShapeDtypeStruct((B, nl), jnp.float32),
    )(z, gamma, beta, lin_w, lin_b, out_w, out_b)


def kernel(x, batch, conv1_w, conv1_b, convs_w, convs_b, bn_gamma, bn_beta,
           lin_w, lin_b, out_w, out_b):
    keep = jax.random.bernoulli(jax.random.key(1), 0.7, (N * K_NN,))
    keep = keep.astype(jnp.float32).reshape(N, K_NN)
    deg = 1.0 + jnp.sum(keep, axis=1)
    dinv = jax.lax.rsqrt(deg)
    coef = dinv[:, None] * keep

    batch_r = batch.reshape(N, 1)
    batch_c = batch.reshape(1, N)
    m = _build_m(x, batch_r, batch_c, coef, dinv.reshape(N, 1), dinv.reshape(1, N))

    h = _prop_dense(m, _prop(m, x), conv1_w, conv1_b.reshape(1, -1))
    feats = [_pool(h, batch_r)]
    for fidx in range(convs_w.shape[0]):
        h = _prop_dense(m, _prop(m, h), convs_w[fidx], convs_b[fidx].reshape(1, -1))
        feats.append(_pool(h, batch_r))
    z = jnp.concatenate(feats, axis=1)
    out = _mlp(z, bn_gamma.reshape(1, -1), bn_beta.reshape(1, -1),
               lin_w, lin_b, out_w, out_b.reshape(1, -1))
    return out.reshape(-1)


# speedup vs baseline: 5.3018x; 1.0453x over previous
import functools

import jax
import jax.numpy as jnp
from jax.experimental import pallas as pl

N = 4096
K_NN = 100
B = 8
RB = 256
PB = 512

_HIGH = jax.lax.Precision.HIGHEST


def _lrelu(v):
    return jnp.where(v >= 0, v, 0.01 * v)


def _graph_kernel(xr_ref, xc_ref, br_ref, bc_ref, coef_ref, dr_ref, dc_ref, m_ref):
    i = pl.program_id(0)
    xr = xr_ref[...]
    xc = xc_ref[...]
    d2r = jnp.sum(xr * xr, axis=1, keepdims=True)
    d2c = jnp.sum(xc * xc, axis=1, keepdims=True).T
    cross = jax.lax.dot_general(xr, xc, (((1,), (1,)), ((), ())),
                                preferred_element_type=jnp.float32)
    dist = d2r + d2c - 2.0 * cross

    iota_c = jax.lax.broadcasted_iota(jnp.int32, (RB, N), 1)
    rows = i * RB + jax.lax.broadcasted_iota(jnp.int32, (RB, 1), 0)
    same = br_ref[...] == bc_ref[...]
    valid = same & (iota_c != rows)
    inf = jnp.float32(jnp.inf)
    dist = jnp.where(valid, dist, inf)

    coef = coef_ref[...]
    iota_k = jax.lax.broadcasted_iota(jnp.int32, (RB, K_NN), 1)

    def body(j, carry):
        d, macc = carry
        amin = jnp.argmin(d, axis=1)[:, None]
        onehot = iota_c == amin
        cj = jnp.sum(jnp.where(iota_k == j, coef, 0.0), axis=1, keepdims=True)
        macc = macc + jnp.where(onehot, cj, 0.0)
        d = jnp.where(onehot, inf, d)
        return d, macc

    macc = jnp.zeros((RB, N), jnp.float32)
    _, macc = jax.lax.fori_loop(0, K_NN, body, (dist, macc))

    self_onehot = iota_c == rows
    macc = macc + jnp.where(self_onehot, dr_ref[...], 0.0)
    m_ref[...] = macc * dc_ref[...]


def _build_m(x, batch_r, batch_c, coef, dinv_r, dinv_c):
    grid = (N // RB,)
    return pl.pallas_call(
        _graph_kernel,
        grid=grid,
        in_specs=[
            pl.BlockSpec((RB, 32), lambda i: (i, 0)),
            pl.BlockSpec((N, 32), lambda i: (0, 0)),
            pl.BlockSpec((RB, 1), lambda i: (i, 0)),
            pl.BlockSpec((1, N), lambda i: (0, 0)),
            pl.BlockSpec((RB, K_NN), lambda i: (i, 0)),
            pl.BlockSpec((RB, 1), lambda i: (i, 0)),
            pl.BlockSpec((1, N), lambda i: (0, 0)),
        ],
        out_specs=pl.BlockSpec((RB, N), lambda i: (i, 0)),
        out_shape=jax.ShapeDtypeStruct((N, N), jnp.float32),
    )(x, x, batch_r, batch_c, coef, dinv_r, dinv_c)


def _prop_kernel(m_ref, h_ref, o_ref):
    o_ref[...] = jax.lax.dot_general(
        m_ref[...], h_ref[...], (((1,), (0,)), ((), ())),
        precision=_HIGH, preferred_element_type=jnp.float32)


def _prop(m, h):
    f = h.shape[1]
    return pl.pallas_call(
        _prop_kernel,
        grid=(N // PB,),
        in_specs=[
            pl.BlockSpec((PB, N), lambda i: (i, 0)),
            pl.BlockSpec((N, f), lambda i: (0, 0)),
        ],
        out_specs=pl.BlockSpec((PB, f), lambda i: (i, 0)),
        out_shape=jax.ShapeDtypeStruct((N, f), jnp.float32),
    )(m, h)


def _prop_dense_kernel(m_ref, h_ref, w_ref, b_ref, o_ref):
    t = jax.lax.dot_general(m_ref[...], h_ref[...], (((1,), (0,)), ((), ())),
                            precision=_HIGH, preferred_element_type=jnp.float32)
    z = jax.lax.dot_general(t, w_ref[...], (((1,), (0,)), ((), ())),
                            precision=_HIGH, preferred_element_type=jnp.float32)
    o_ref[...] = _lrelu(z + b_ref[...])


def _prop_dense(m, h, w, b):
    f_in = h.shape[1]
    f_out = w.shape[1]
    return pl.pallas_call(
        _prop_dense_kernel,
        grid=(N // PB,),
        in_specs=[
            pl.BlockSpec((PB, N), lambda i: (i, 0)),
            pl.BlockSpec((N, f_in), lambda i: (0, 0)),
            pl.BlockSpec((f_in, f_out), lambda i: (0, 0)),
            pl.BlockSpec((1, f_out), lambda i: (0, 0)),
        ],
        out_specs=pl.BlockSpec((PB, f_out), lambda i: (i, 0)),
        out_shape=jax.ShapeDtypeStruct((N, f_out), jnp.float32),
    )(m, h, w, b)


def _pool_kernel(h_ref, b_ref, o_ref):
    h = h_ref[...]
    bt = b_ref[...]
    ninf = jnp.float32(-jnp.inf)
    rows = []
    for g in range(B):
        mask = bt == g
        s = jnp.sum(jnp.where(mask, h, 0.0), axis=0)
        cnt = jnp.sum(jnp.where(mask, 1.0, 0.0))
        mean = s / jnp.maximum(cnt, 1.0)
        mx = jnp.max(jnp.where(mask, h, ninf), axis=0)
        mx = jnp.where(jnp.isfinite(mx), mx, 0.0)
        rows.append(jnp.concatenate([mean, mx]))
    o_ref[...] = jnp.stack(rows)


def _pool(h, batch_r):
    f = h.shape[1]
    return pl.pallas_call(
        _pool_kernel,
        in_specs=[
            pl.BlockSpec((N, f), lambda: (0, 0)),
            pl.BlockSpec((N, 1), lambda: (0, 0)),
        ],
        out_specs=pl.BlockSpec((B, 2 * f), lambda: (0, 0)),
        out_shape=jax.ShapeDtypeStruct((B, 2 * f), jnp.float32),
    )(h, batch_r)


def _mlp_kernel(z_ref, g_ref, be_ref, lw_ref, lb_ref, ow_ref, ob_ref, o_ref):
    z = z_ref[...]
    mu = jnp.mean(z, axis=0, keepdims=True)
    var = jnp.mean((z - mu) ** 2, axis=0, keepdims=True)
    z = (z - mu) / jnp.sqrt(var + 1e-5) * g_ref[...] + be_ref[...]
    for g in range(lw_ref.shape[0]):
        z = _lrelu(jax.lax.dot_general(
            z, lw_ref[g], (((1,), (0,)), ((), ())),
            precision=_HIGH, preferred_element_type=jnp.float32) + lb_ref[g][None, :])
    o_ref[...] = jax.lax.dot_general(
        z, ow_ref[...], (((1,), (0,)), ((), ())),
        precision=_HIGH, preferred_element_type=jnp.float32) + ob_ref[...]


def _mlp(z, gamma, beta, lin_w, lin_b, out_w, out_b):
    f = z.shape[1]
    nl = out_w.shape[1]
    return pl.pallas_call(
        _mlp_kernel,
        in_specs=[
            pl.BlockSpec((B, f), lambda: (0, 0)),
            pl.BlockSpec((1, f), lambda: (0, 0)),
            pl.BlockSpec((1, f), lambda: (0, 0)),
            pl.BlockSpec(lin_w.shape, lambda: (0, 0, 0)),
            pl.BlockSpec(lin_b.shape, lambda: (0, 0)),
            pl.BlockSpec(out_w.shape, lambda: (0, 0)),
            pl.BlockSpec((1, nl), lambda: (0, 0)),
        ],
        out_specs=pl.BlockSpec((B, nl), lambda: (0, 0)),
        out_shape=jax.ShapeDtypeStruct((B, nl), jnp.float32),
    )(z, gamma, beta, lin_w, lin_b, out_w, out_b)


def kernel(x, batch, conv1_w, conv1_b, convs_w, convs_b, bn_gamma, bn_beta,
           lin_w, lin_b, out_w, out_b):
    keep = jax.random.bernoulli(jax.random.key(1), 0.7, (N * K_NN,))
    keep = keep.astype(jnp.float32).reshape(N, K_NN)
    deg = 1.0 + jnp.sum(keep, axis=1)
    dinv = jax.lax.rsqrt(deg)
    coef = dinv[:, None] * keep

    batch_r = batch.reshape(N, 1)
    batch_c = batch.reshape(1, N)
    m = _build_m(x, batch_r, batch_c, coef, dinv.reshape(N, 1), dinv.reshape(1, N))

    h = _prop_dense(m, _prop(m, x), conv1_w, conv1_b.reshape(1, -1))
    feats = [_pool(h, batch_r)]
    for fidx in range(convs_w.shape[0]):
        h = _prop_dense(m, _prop(m, h), convs_w[fidx], convs_b[fidx].reshape(1, -1))
        feats.append(_pool(h, batch_r))
    z = jnp.concatenate(feats, axis=1)
    out = _mlp(z, bn_gamma.reshape(1, -1), bn_beta.reshape(1, -1),
               lin_w, lin_b, out_w, out_b.reshape(1, -1))
    return out.reshape(-1)


# speedup vs baseline: 11.0050x; 2.0757x over previous
import functools

import jax
import jax.numpy as jnp
from jax.experimental import pallas as pl

N = 4096
K_NN = 100
B = 8
RB = 256
PB = 512
W_FAST = 1536

_HIGH = jax.lax.Precision.HIGHEST


def _lrelu(v):
    return jnp.where(v >= 0, v, 0.01 * v)


def _graph_kernel(w_cols, xr_ref, xc_ref, br_ref, bc_ref, coef_ref, dr_ref,
                  dc_ref, m_ref):
    i = pl.program_id(0)
    xr = xr_ref[...]
    if w_cols == N:
        w = 0
    else:
        b0 = br_ref[0, 0]
        iota_full = jax.lax.broadcasted_iota(jnp.int32, (1, N), 1)
        first = jnp.min(jnp.where(bc_ref[...] == b0, iota_full, N))
        w = jnp.minimum((first // 128) * 128, N - w_cols)
    xc = xc_ref[pl.ds(w, w_cols), :]
    bc = bc_ref[:, pl.ds(w, w_cols)]
    dc = dc_ref[:, pl.ds(w, w_cols)]
    d2r = jnp.sum(xr * xr, axis=1, keepdims=True)
    d2c = jnp.sum(xc * xc, axis=1, keepdims=True).T
    cross = jax.lax.dot_general(xr, xc, (((1,), (1,)), ((), ())),
                                preferred_element_type=jnp.float32)
    dist = d2r + d2c - 2.0 * cross

    iota_c = w + jax.lax.broadcasted_iota(jnp.int32, (RB, w_cols), 1)
    rows = i * RB + jax.lax.broadcasted_iota(jnp.int32, (RB, 1), 0)
    same = br_ref[...] == bc
    valid = same & (iota_c != rows)
    inf = jnp.float32(jnp.inf)
    dist = jnp.where(valid, dist, inf)

    coef = coef_ref[...]
    iota_k = jax.lax.broadcasted_iota(jnp.int32, (RB, K_NN), 1)

    def body(j, carry):
        d, macc = carry
        amin = w + jnp.argmin(d, axis=1)[:, None]
        onehot = iota_c == amin
        cj = jnp.sum(jnp.where(iota_k == j, coef, 0.0), axis=1, keepdims=True)
        macc = macc + jnp.where(onehot, cj, 0.0)
        d = jnp.where(onehot, inf, d)
        return d, macc

    macc = jnp.zeros((RB, w_cols), jnp.float32)
    _, macc = jax.lax.fori_loop(0, K_NN, body, (dist, macc))

    self_onehot = iota_c == rows
    macc = macc + jnp.where(self_onehot, dr_ref[...], 0.0)
    if w_cols == N:
        m_ref[...] = macc * dc
    else:
        m_ref[...] = jnp.zeros((RB, N), jnp.float32)
        m_ref[:, pl.ds(w, w_cols)] = macc * dc


def _build_m(w_cols, x, batch_r, batch_c, coef, dinv_r, dinv_c):
    grid = (N // RB,)
    return pl.pallas_call(
        functools.partial(_graph_kernel, w_cols),
        grid=grid,
        in_specs=[
            pl.BlockSpec((RB, 32), lambda i: (i, 0)),
            pl.BlockSpec((N, 32), lambda i: (0, 0)),
            pl.BlockSpec((RB, 1), lambda i: (i, 0)),
            pl.BlockSpec((1, N), lambda i: (0, 0)),
            pl.BlockSpec((RB, K_NN), lambda i: (i, 0)),
            pl.BlockSpec((RB, 1), lambda i: (i, 0)),
            pl.BlockSpec((1, N), lambda i: (0, 0)),
        ],
        out_specs=pl.BlockSpec((RB, N), lambda i: (i, 0)),
        out_shape=jax.ShapeDtypeStruct((N, N), jnp.float32),
    )(x, x, batch_r, batch_c, coef, dinv_r, dinv_c)


def _prop_kernel(m_ref, h_ref, o_ref):
    o_ref[...] = jax.lax.dot_general(
        m_ref[...], h_ref[...], (((1,), (0,)), ((), ())),
        precision=_HIGH, preferred_element_type=jnp.float32)


def _prop(m, h):
    f = h.shape[1]
    return pl.pallas_call(
        _prop_kernel,
        grid=(N // PB,),
        in_specs=[
            pl.BlockSpec((PB, N), lambda i: (i, 0)),
            pl.BlockSpec((N, f), lambda i: (0, 0)),
        ],
        out_specs=pl.BlockSpec((PB, f), lambda i: (i, 0)),
        out_shape=jax.ShapeDtypeStruct((N, f), jnp.float32),
    )(m, h)


def _prop_dense_kernel(m_ref, h_ref, w_ref, b_ref, o_ref):
    t = jax.lax.dot_general(m_ref[...], h_ref[...], (((1,), (0,)), ((), ())),
                            precision=_HIGH, preferred_element_type=jnp.float32)
    z = jax.lax.dot_general(t, w_ref[...], (((1,), (0,)), ((), ())),
                            precision=_HIGH, preferred_element_type=jnp.float32)
    o_ref[...] = _lrelu(z + b_ref[...])


def _prop_dense(m, h, w, b):
    f_in = h.shape[1]
    f_out = w.shape[1]
    return pl.pallas_call(
        _prop_dense_kernel,
        grid=(N // PB,),
        in_specs=[
            pl.BlockSpec((PB, N), lambda i: (i, 0)),
            pl.BlockSpec((N, f_in), lambda i: (0, 0)),
            pl.BlockSpec((f_in, f_out), lambda i: (0, 0)),
            pl.BlockSpec((1, f_out), lambda i: (0, 0)),
        ],
        out_specs=pl.BlockSpec((PB, f_out), lambda i: (i, 0)),
        out_shape=jax.ShapeDtypeStruct((N, f_out), jnp.float32),
    )(m, h, w, b)


def _pool_kernel(h_ref, b_ref, o_ref):
    h = h_ref[...]
    bt = b_ref[...]
    ninf = jnp.float32(-jnp.inf)
    rows = []
    for g in range(B):
        mask = bt == g
        s = jnp.sum(jnp.where(mask, h, 0.0), axis=0)
        cnt = jnp.sum(jnp.where(mask, 1.0, 0.0))
        mean = s / jnp.maximum(cnt, 1.0)
        mx = jnp.max(jnp.where(mask, h, ninf), axis=0)
        mx = jnp.where(jnp.isfinite(mx), mx, 0.0)
        rows.append(jnp.concatenate([mean, mx]))
    o_ref[...] = jnp.stack(rows)


def _pool(h, batch_r):
    f = h.shape[1]
    return pl.pallas_call(
        _pool_kernel,
        in_specs=[
            pl.BlockSpec((N, f), lambda: (0, 0)),
            pl.BlockSpec((N, 1), lambda: (0, 0)),
        ],
        out_specs=pl.BlockSpec((B, 2 * f), lambda: (0, 0)),
        out_shape=jax.ShapeDtypeStruct((B, 2 * f), jnp.float32),
    )(h, batch_r)


def _mlp_kernel(z_ref, g_ref, be_ref, lw_ref, lb_ref, ow_ref, ob_ref, o_ref):
    z = z_ref[...]
    mu = jnp.mean(z, axis=0, keepdims=True)
    var = jnp.mean((z - mu) ** 2, axis=0, keepdims=True)
    z = (z - mu) / jnp.sqrt(var + 1e-5) * g_ref[...] + be_ref[...]
    for g in range(lw_ref.shape[0]):
        z = _lrelu(jax.lax.dot_general(
            z, lw_ref[g], (((1,), (0,)), ((), ())),
            precision=_HIGH, preferred_element_type=jnp.float32) + lb_ref[g][None, :])
    o_ref[...] = jax.lax.dot_general(
        z, ow_ref[...], (((1,), (0,)), ((), ())),
        precision=_HIGH, preferred_element_type=jnp.float32) + ob_ref[...]


def _mlp(z, gamma, beta, lin_w, lin_b, out_w, out_b):
    f = z.shape[1]
    nl = out_w.shape[1]
    return pl.pallas_call(
        _mlp_kernel,
        in_specs=[
            pl.BlockSpec((B, f), lambda: (0, 0)),
            pl.BlockSpec((1, f), lambda: (0, 0)),
            pl.BlockSpec((1, f), lambda: (0, 0)),
            pl.BlockSpec(lin_w.shape, lambda: (0, 0, 0)),
            pl.BlockSpec(lin_b.shape, lambda: (0, 0)),
            pl.BlockSpec(out_w.shape, lambda: (0, 0)),
            pl.BlockSpec((1, nl), lambda: (0, 0)),
        ],
        out_specs=pl.BlockSpec((B, nl), lambda: (0, 0)),
        out_shape=jax.ShapeDtypeStruct((B, nl), jnp.float32),
    )(z, gamma, beta, lin_w, lin_b, out_w, out_b)


def kernel(x, batch, conv1_w, conv1_b, convs_w, convs_b, bn_gamma, bn_beta,
           lin_w, lin_b, out_w, out_b):
    keep = jax.random.bernoulli(jax.random.key(1), 0.7, (N * K_NN,))
    keep = keep.astype(jnp.float32).reshape(N, K_NN)
    deg = 1.0 + jnp.sum(keep, axis=1)
    dinv = jax.lax.rsqrt(deg)
    coef = dinv[:, None] * keep

    batch_r = batch.reshape(N, 1)
    batch_c = batch.reshape(1, N)
    dinv_r = dinv.reshape(N, 1)
    dinv_c = dinv.reshape(1, N)

    firsts = batch[::RB]
    lasts = batch[RB - 1::RB]
    seg_lo = (jnp.searchsorted(batch, firsts, side="left") // 128) * 128
    seg_hi = jnp.searchsorted(batch, lasts, side="right")
    fits = jnp.max(seg_hi - seg_lo) <= W_FAST
    m = jax.lax.cond(
        fits,
        lambda a: _build_m(W_FAST, *a),
        lambda a: _build_m(N, *a),
        (x, batch_r, batch_c, coef, dinv_r, dinv_c))

    h = _prop_dense(m, _prop(m, x), conv1_w, conv1_b.reshape(1, -1))
    feats = [_pool(h, batch_r)]
    for fidx in range(convs_w.shape[0]):
        h = _prop_dense(m, _prop(m, h), convs_w[fidx], convs_b[fidx].reshape(1, -1))
        feats.append(_pool(h, batch_r))
    z = jnp.concatenate(feats, axis=1)
    out = _mlp(z, bn_gamma.reshape(1, -1), bn_beta.reshape(1, -1),
               lin_w, lin_b, out_w, out_b.reshape(1, -1))
    return out.reshape(-1)


# speedup vs baseline: 12.0663x; 1.0964x over previous
import functools

import jax
import jax.numpy as jnp
from jax.experimental import pallas as pl

N = 4096
K_NN = 100
B = 8
RB = 256
PB = 512
W_FAST = 1280

_HIGH = jax.lax.Precision.HIGHEST


def _lrelu(v):
    return jnp.where(v >= 0, v, 0.01 * v)


def _graph_kernel(w_cols, xr_ref, xc_ref, br_ref, bc_ref, coef_ref, dr_ref,
                  dc_ref, m_ref):
    i = pl.program_id(0)
    xr = xr_ref[...]
    if w_cols == N:
        w = 0
    else:
        b0 = br_ref[0, 0]
        iota_full = jax.lax.broadcasted_iota(jnp.int32, (1, N), 1)
        first = jnp.min(jnp.where(bc_ref[...] == b0, iota_full, N))
        w = jnp.minimum((first // 128) * 128, N - w_cols)
    xc = xc_ref[pl.ds(w, w_cols), :]
    bc = bc_ref[:, pl.ds(w, w_cols)]
    dc = dc_ref[:, pl.ds(w, w_cols)]
    d2r = jnp.sum(xr * xr, axis=1, keepdims=True)
    d2c = jnp.sum(xc * xc, axis=1, keepdims=True).T
    cross = jax.lax.dot_general(xr, xc, (((1,), (1,)), ((), ())),
                                preferred_element_type=jnp.float32)
    dist = d2r + d2c - 2.0 * cross

    iota_c = w + jax.lax.broadcasted_iota(jnp.int32, (RB, w_cols), 1)
    rows = i * RB + jax.lax.broadcasted_iota(jnp.int32, (RB, 1), 0)
    same = br_ref[...] == bc
    valid = same & (iota_c != rows)
    inf = jnp.float32(jnp.inf)
    dist = jnp.where(valid, dist, inf)

    coef = coef_ref[...]
    iota_k = jax.lax.broadcasted_iota(jnp.int32, (RB, K_NN), 1)

    def body(j, carry):
        d, macc = carry
        amin = w + jnp.argmin(d, axis=1)[:, None]
        onehot = iota_c == amin
        cj = jnp.sum(jnp.where(iota_k == j, coef, 0.0), axis=1, keepdims=True)
        macc = macc + jnp.where(onehot, cj, 0.0)
        d = jnp.where(onehot, inf, d)
        return d, macc

    macc = jnp.zeros((RB, w_cols), jnp.float32)
    _, macc = jax.lax.fori_loop(0, K_NN, body, (dist, macc))

    self_onehot = iota_c == rows
    macc = macc + jnp.where(self_onehot, dr_ref[...], 0.0)
    if w_cols == N:
        m_ref[...] = macc * dc
    else:
        m_ref[...] = jnp.zeros((RB, N), jnp.float32)
        m_ref[:, pl.ds(w, w_cols)] = macc * dc


def _build_m(w_cols, x, batch_r, batch_c, coef, dinv_r, dinv_c):
    grid = (N // RB,)
    return pl.pallas_call(
        functools.partial(_graph_kernel, w_cols),
        grid=grid,
        in_specs=[
            pl.BlockSpec((RB, 32), lambda i: (i, 0)),
            pl.BlockSpec((N, 32), lambda i: (0, 0)),
            pl.BlockSpec((RB, 1), lambda i: (i, 0)),
            pl.BlockSpec((1, N), lambda i: (0, 0)),
            pl.BlockSpec((RB, K_NN), lambda i: (i, 0)),
            pl.BlockSpec((RB, 1), lambda i: (i, 0)),
            pl.BlockSpec((1, N), lambda i: (0, 0)),
        ],
        out_specs=pl.BlockSpec((RB, N), lambda i: (i, 0)),
        out_shape=jax.ShapeDtypeStruct((N, N), jnp.float32),
    )(x, x, batch_r, batch_c, coef, dinv_r, dinv_c)


def _prop_kernel(m_ref, h_ref, o_ref):
    o_ref[...] = jax.lax.dot_general(
        m_ref[...], h_ref[...], (((1,), (0,)), ((), ())),
        precision=_HIGH, preferred_element_type=jnp.float32)


def _prop(m, h):
    f = h.shape[1]
    return pl.pallas_call(
        _prop_kernel,
        grid=(N // PB,),
        in_specs=[
            pl.BlockSpec((PB, N), lambda i: (i, 0)),
            pl.BlockSpec((N, f), lambda i: (0, 0)),
        ],
        out_specs=pl.BlockSpec((PB, f), lambda i: (i, 0)),
        out_shape=jax.ShapeDtypeStruct((N, f), jnp.float32),
    )(m, h)


def _prop_dense_kernel(m_ref, h_ref, w_ref, b_ref, o_ref):
    t = jax.lax.dot_general(m_ref[...], h_ref[...], (((1,), (0,)), ((), ())),
                            precision=_HIGH, preferred_element_type=jnp.float32)
    z = jax.lax.dot_general(t, w_ref[...], (((1,), (0,)), ((), ())),
                            precision=_HIGH, preferred_element_type=jnp.float32)
    o_ref[...] = _lrelu(z + b_ref[...])


def _prop_dense(m, h, w, b):
    f_in = h.shape[1]
    f_out = w.shape[1]
    return pl.pallas_call(
        _prop_dense_kernel,
        grid=(N // PB,),
        in_specs=[
            pl.BlockSpec((PB, N), lambda i: (i, 0)),
            pl.BlockSpec((N, f_in), lambda i: (0, 0)),
            pl.BlockSpec((f_in, f_out), lambda i: (0, 0)),
            pl.BlockSpec((1, f_out), lambda i: (0, 0)),
        ],
        out_specs=pl.BlockSpec((PB, f_out), lambda i: (i, 0)),
        out_shape=jax.ShapeDtypeStruct((N, f_out), jnp.float32),
    )(m, h, w, b)


def _pool_kernel(h_ref, b_ref, o_ref):
    h = h_ref[...]
    bt = b_ref[...]
    ninf = jnp.float32(-jnp.inf)
    rows = []
    for g in range(B):
        mask = bt == g
        s = jnp.sum(jnp.where(mask, h, 0.0), axis=0)
        cnt = jnp.sum(jnp.where(mask, 1.0, 0.0))
        mean = s / jnp.maximum(cnt, 1.0)
        mx = jnp.max(jnp.where(mask, h, ninf), axis=0)
        mx = jnp.where(jnp.isfinite(mx), mx, 0.0)
        rows.append(jnp.concatenate([mean, mx]))
    o_ref[...] = jnp.stack(rows)


def _pool(h, batch_r):
    f = h.shape[1]
    return pl.pallas_call(
        _pool_kernel,
        in_specs=[
            pl.BlockSpec((N, f), lambda: (0, 0)),
            pl.BlockSpec((N, 1), lambda: (0, 0)),
        ],
        out_specs=pl.BlockSpec((B, 2 * f), lambda: (0, 0)),
        out_shape=jax.ShapeDtypeStruct((B, 2 * f), jnp.float32),
    )(h, batch_r)


def _mlp_kernel(z_ref, g_ref, be_ref, lw_ref, lb_ref, ow_ref, ob_ref, o_ref):
    z = z_ref[...]
    mu = jnp.mean(z, axis=0, keepdims=True)
    var = jnp.mean((z - mu) ** 2, axis=0, keepdims=True)
    z = (z - mu) / jnp.sqrt(var + 1e-5) * g_ref[...] + be_ref[...]
    for g in range(lw_ref.shape[0]):
        z = _lrelu(jax.lax.dot_general(
            z, lw_ref[g], (((1,), (0,)), ((), ())),
            precision=_HIGH, preferred_element_type=jnp.float32) + lb_ref[g][None, :])
    o_ref[...] = jax.lax.dot_general(
        z, ow_ref[...], (((1,), (0,)), ((), ())),
        precision=_HIGH, preferred_element_type=jnp.float32) + ob_ref[...]


def _mlp(z, gamma, beta, lin_w, lin_b, out_w, out_b):
    f = z.shape[1]
    nl = out_w.shape[1]
    return pl.pallas_call(
        _mlp_kernel,
        in_specs=[
            pl.BlockSpec((B, f), lambda: (0, 0)),
            pl.BlockSpec((1, f), lambda: (0, 0)),
            pl.BlockSpec((1, f), lambda: (0, 0)),
            pl.BlockSpec(lin_w.shape, lambda: (0, 0, 0)),
            pl.BlockSpec(lin_b.shape, lambda: (0, 0)),
            pl.BlockSpec(out_w.shape, lambda: (0, 0)),
            pl.BlockSpec((1, nl), lambda: (0, 0)),
        ],
        out_specs=pl.BlockSpec((B, nl), lambda: (0, 0)),
        out_shape=jax.ShapeDtypeStruct((B, nl), jnp.float32),
    )(z, gamma, beta, lin_w, lin_b, out_w, out_b)


def kernel(x, batch, conv1_w, conv1_b, convs_w, convs_b, bn_gamma, bn_beta,
           lin_w, lin_b, out_w, out_b):
    keep = jax.random.bernoulli(jax.random.key(1), 0.7, (N * K_NN,))
    keep = keep.astype(jnp.float32).reshape(N, K_NN)
    deg = 1.0 + jnp.sum(keep, axis=1)
    dinv = jax.lax.rsqrt(deg)
    coef = dinv[:, None] * keep

    batch_r = batch.reshape(N, 1)
    batch_c = batch.reshape(1, N)
    dinv_r = dinv.reshape(N, 1)
    dinv_c = dinv.reshape(1, N)

    firsts = batch[::RB]
    lasts = batch[RB - 1::RB]
    seg_lo = (jnp.searchsorted(batch, firsts, side="left") // 128) * 128
    seg_hi = jnp.searchsorted(batch, lasts, side="right")
    fits = jnp.max(seg_hi - seg_lo) <= W_FAST
    m = jax.lax.cond(
        fits,
        lambda a: _build_m(W_FAST, *a),
        lambda a: _build_m(N, *a),
        (x, batch_r, batch_c, coef, dinv_r, dinv_c))

    h = _prop_dense(m, _prop(m, x), conv1_w, conv1_b.reshape(1, -1))
    feats = [_pool(h, batch_r)]
    for fidx in range(convs_w.shape[0]):
        h = _prop_dense(m, _prop(m, h), convs_w[fidx], convs_b[fidx].reshape(1, -1))
        feats.append(_pool(h, batch_r))
    z = jnp.concatenate(feats, axis=1)
    out = _mlp(z, bn_gamma.reshape(1, -1), bn_beta.reshape(1, -1),
               lin_w, lin_b, out_w, out_b.reshape(1, -1))
    return out.reshape(-1)


# speedup vs baseline: 13.2010x; 1.0940x over previous
import functools

import jax
import jax.numpy as jnp
from jax.experimental import pallas as pl

N = 4096
K_NN = 100
B = 8
RB = 256
PB = 512
W_FAST = 1280
W_NARROW = 768

_HIGH = jax.lax.Precision.HIGHEST


def _lrelu(v):
    return jnp.where(v >= 0, v, 0.01 * v)


def _graph_kernel(w_cols, xr_ref, xc_ref, br_ref, bc_ref, coef_ref, dr_ref,
                  dc_ref, m_ref):
    i = pl.program_id(0)
    if w_cols == N:
        _graph_body(N, 0, i, xr_ref, xc_ref, br_ref, bc_ref, coef_ref, dr_ref,
                    dc_ref, m_ref)
        return
    b0 = br_ref[0, 0]
    b1 = br_ref[RB - 1, 0]
    iota_full = jax.lax.broadcasted_iota(jnp.int32, (1, N), 1)
    first = jnp.min(jnp.where(bc_ref[...] == b0, iota_full, N))
    last = jnp.max(jnp.where(bc_ref[...] == b1, iota_full + 1, 0))
    w = (first // 128) * 128
    w_narrow = jnp.minimum(w, N - W_NARROW)
    narrow = last - w_narrow <= W_NARROW

    @pl.when(narrow)
    def _():
        _graph_body(W_NARROW, w_narrow, i, xr_ref, xc_ref, br_ref, bc_ref,
                    coef_ref, dr_ref, dc_ref, m_ref)

    @pl.when(jnp.logical_not(narrow))
    def _():
        _graph_body(w_cols, jnp.minimum(w, N - w_cols), i, xr_ref, xc_ref,
                    br_ref, bc_ref, coef_ref, dr_ref, dc_ref, m_ref)


def _graph_body(w_cols, w, i, xr_ref, xc_ref, br_ref, bc_ref, coef_ref,
                dr_ref, dc_ref, m_ref):
    xr = xr_ref[...]
    xc = xc_ref[pl.ds(w, w_cols), :]
    bc = bc_ref[:, pl.ds(w, w_cols)]
    dc = dc_ref[:, pl.ds(w, w_cols)]
    d2r = jnp.sum(xr * xr, axis=1, keepdims=True)
    d2c = jnp.sum(xc * xc, axis=1, keepdims=True).T
    cross = jax.lax.dot_general(xr, xc, (((1,), (1,)), ((), ())),
                                preferred_element_type=jnp.float32)
    dist = d2r + d2c - 2.0 * cross

    iota_c = w + jax.lax.broadcasted_iota(jnp.int32, (RB, w_cols), 1)
    rows = i * RB + jax.lax.broadcasted_iota(jnp.int32, (RB, 1), 0)
    same = br_ref[...] == bc
    valid = same & (iota_c != rows)
    inf = jnp.float32(jnp.inf)
    dist = jnp.where(valid, dist, inf)

    coef = coef_ref[...]
    iota_k = jax.lax.broadcasted_iota(jnp.int32, (RB, K_NN), 1)

    def body(j, carry):
        d, macc = carry
        amin = w + jnp.argmin(d, axis=1)[:, None]
        onehot = iota_c == amin
        cj = jnp.sum(jnp.where(iota_k == j, coef, 0.0), axis=1, keepdims=True)
        macc = macc + jnp.where(onehot, cj, 0.0)
        d = jnp.where(onehot, inf, d)
        return d, macc

    macc = jnp.zeros((RB, w_cols), jnp.float32)
    _, macc = jax.lax.fori_loop(0, K_NN, body, (dist, macc))

    self_onehot = iota_c == rows
    macc = macc + jnp.where(self_onehot, dr_ref[...], 0.0)
    if w_cols == N:
        m_ref[...] = macc * dc
    else:
        m_ref[...] = jnp.zeros((RB, N), jnp.float32)
        m_ref[:, pl.ds(w, w_cols)] = macc * dc


def _build_m(w_cols, x, batch_r, batch_c, coef, dinv_r, dinv_c):
    grid = (N // RB,)
    return pl.pallas_call(
        functools.partial(_graph_kernel, w_cols),
        grid=grid,
        in_specs=[
            pl.BlockSpec((RB, 32), lambda i: (i, 0)),
            pl.BlockSpec((N, 32), lambda i: (0, 0)),
            pl.BlockSpec((RB, 1), lambda i: (i, 0)),
            pl.BlockSpec((1, N), lambda i: (0, 0)),
            pl.BlockSpec((RB, K_NN), lambda i: (i, 0)),
            pl.BlockSpec((RB, 1), lambda i: (i, 0)),
            pl.BlockSpec((1, N), lambda i: (0, 0)),
        ],
        out_specs=pl.BlockSpec((RB, N), lambda i: (i, 0)),
        out_shape=jax.ShapeDtypeStruct((N, N), jnp.float32),
    )(x, x, batch_r, batch_c, coef, dinv_r, dinv_c)


def _prop_kernel(m_ref, h_ref, o_ref):
    o_ref[...] = jax.lax.dot_general(
        m_ref[...], h_ref[...], (((1,), (0,)), ((), ())),
        precision=_HIGH, preferred_element_type=jnp.float32)


def _prop(m, h):
    f = h.shape[1]
    return pl.pallas_call(
        _prop_kernel,
        grid=(N // PB,),
        in_specs=[
            pl.BlockSpec((PB, N), lambda i: (i, 0)),
            pl.BlockSpec((N, f), lambda i: (0, 0)),
        ],
        out_specs=pl.BlockSpec((PB, f), lambda i: (i, 0)),
        out_shape=jax.ShapeDtypeStruct((N, f), jnp.float32),
    )(m, h)


def _prop_dense_kernel(m_ref, h_ref, w_ref, b_ref, o_ref):
    t = jax.lax.dot_general(m_ref[...], h_ref[...], (((1,), (0,)), ((), ())),
                            precision=_HIGH, preferred_element_type=jnp.float32)
    z = jax.lax.dot_general(t, w_ref[...], (((1,), (0,)), ((), ())),
                            precision=_HIGH, preferred_element_type=jnp.float32)
    o_ref[...] = _lrelu(z + b_ref[...])


def _prop_dense(m, h, w, b):
    f_in = h.shape[1]
    f_out = w.shape[1]
    return pl.pallas_call(
        _prop_dense_kernel,
        grid=(N // PB,),
        in_specs=[
            pl.BlockSpec((PB, N), lambda i: (i, 0)),
            pl.BlockSpec((N, f_in), lambda i: (0, 0)),
            pl.BlockSpec((f_in, f_out), lambda i: (0, 0)),
            pl.BlockSpec((1, f_out), lambda i: (0, 0)),
        ],
        out_specs=pl.BlockSpec((PB, f_out), lambda i: (i, 0)),
        out_shape=jax.ShapeDtypeStruct((N, f_out), jnp.float32),
    )(m, h, w, b)


def _pool_kernel(h_ref, b_ref, o_ref):
    h = h_ref[...]
    bt = b_ref[...]
    ninf = jnp.float32(-jnp.inf)
    rows = []
    for g in range(B):
        mask = bt == g
        s = jnp.sum(jnp.where(mask, h, 0.0), axis=0)
        cnt = jnp.sum(jnp.where(mask, 1.0, 0.0))
        mean = s / jnp.maximum(cnt, 1.0)
        mx = jnp.max(jnp.where(mask, h, ninf), axis=0)
        mx = jnp.where(jnp.isfinite(mx), mx, 0.0)
        rows.append(jnp.concatenate([mean, mx]))
    o_ref[...] = jnp.stack(rows)


def _pool(h, batch_r):
    f = h.shape[1]
    return pl.pallas_call(
        _pool_kernel,
        in_specs=[
            pl.BlockSpec((N, f), lambda: (0, 0)),
            pl.BlockSpec((N, 1), lambda: (0, 0)),
        ],
        out_specs=pl.BlockSpec((B, 2 * f), lambda: (0, 0)),
        out_shape=jax.ShapeDtypeStruct((B, 2 * f), jnp.float32),
    )(h, batch_r)


def _mlp_kernel(z_ref, g_ref, be_ref, lw_ref, lb_ref, ow_ref, ob_ref, o_ref):
    z = z_ref[...]
    mu = jnp.mean(z, axis=0, keepdims=True)
    var = jnp.mean((z - mu) ** 2, axis=0, keepdims=True)
    z = (z - mu) / jnp.sqrt(var + 1e-5) * g_ref[...] + be_ref[...]
    for g in range(lw_ref.shape[0]):
        z = _lrelu(jax.lax.dot_general(
            z, lw_ref[g], (((1,), (0,)), ((), ())),
            precision=_HIGH, preferred_element_type=jnp.float32) + lb_ref[g][None, :])
    o_ref[...] = jax.lax.dot_general(
        z, ow_ref[...], (((1,), (0,)), ((), ())),
        precision=_HIGH, preferred_element_type=jnp.float32) + ob_ref[...]


def _mlp(z, gamma, beta, lin_w, lin_b, out_w, out_b):
    f = z.shape[1]
    nl = out_w.shape[1]
    return pl.pallas_call(
        _mlp_kernel,
        in_specs=[
            pl.BlockSpec((B, f), lambda: (0, 0)),
            pl.BlockSpec((1, f), lambda: (0, 0)),
            pl.BlockSpec((1, f), lambda: (0, 0)),
            pl.BlockSpec(lin_w.shape, lambda: (0, 0, 0)),
            pl.BlockSpec(lin_b.shape, lambda: (0, 0)),
            pl.BlockSpec(out_w.shape, lambda: (0, 0)),
            pl.BlockSpec((1, nl), lambda: (0, 0)),
        ],
        out_specs=pl.BlockSpec((B, nl), lambda: (0, 0)),
        out_shape=jax.ShapeDtypeStruct((B, nl), jnp.float32),
    )(z, gamma, beta, lin_w, lin_b, out_w, out_b)


def kernel(x, batch, conv1_w, conv1_b, convs_w, convs_b, bn_gamma, bn_beta,
           lin_w, lin_b, out_w, out_b):
    keep = jax.random.bernoulli(jax.random.key(1), 0.7, (N * K_NN,))
    keep = keep.astype(jnp.float32).reshape(N, K_NN)
    deg = 1.0 + jnp.sum(keep, axis=1)
    dinv = jax.lax.rsqrt(deg)
    coef = dinv[:, None] * keep

    batch_r = batch.reshape(N, 1)
    batch_c = batch.reshape(1, N)
    dinv_r = dinv.reshape(N, 1)
    dinv_c = dinv.reshape(1, N)

    firsts = batch[::RB]
    lasts = batch[RB - 1::RB]
    seg_lo = (jnp.searchsorted(batch, firsts, side="left") // 128) * 128
    seg_hi = jnp.searchsorted(batch, lasts, side="right")
    fits = jnp.max(seg_hi - seg_lo) <= W_FAST
    m = jax.lax.cond(
        fits,
        lambda a: _build_m(W_FAST, *a),
        lambda a: _build_m(N, *a),
        (x, batch_r, batch_c, coef, dinv_r, dinv_c))

    h = _prop_dense(m, _prop(m, x), conv1_w, conv1_b.reshape(1, -1))
    feats = [_pool(h, batch_r)]
    for fidx in range(convs_w.shape[0]):
        h = _prop_dense(m, _prop(m, h), convs_w[fidx], convs_b[fidx].reshape(1, -1))
        feats.append(_pool(h, batch_r))
    z = jnp.concatenate(feats, axis=1)
    out = _mlp(z, bn_gamma.reshape(1, -1), bn_beta.reshape(1, -1),
               lin_w, lin_b, out_w, out_b.reshape(1, -1))
    return out.reshape(-1)
